# Initial kernel scaffold; baseline (speedup 1.0000x reference)
import jax
import jax.numpy as jnp
from jax.experimental import pallas as pl


def kernel(x, edge_index, W1, b1, W2, b2, Wl, bl):
    raise NotImplementedError("write your pallas kernel here")


# speedup vs baseline: 20.3535x; 20.3535x over previous
import functools

import jax
import jax.numpy as jnp
from jax import lax
from jax.experimental import pallas as pl
from jax.experimental.pallas import tpu as pltpu
from jax.experimental.pallas import tpu_sc as plsc

N = 10000
E = 320000
F_IN = 128
DIM = 32
C = 10

NC = 2
NS = 16
NW = NC * NS
CH = 128
NCHUNK = E // CH
KMAX = -(-NCHUNK // NW)
RPT = 640
DEGW = 16


def _tile_row0(sid):
    return pl.multiple_of(jnp.minimum(sid * RPT, N - RPT), 8)

_mesh = plsc.VectorSubcoreMesh(core_axis_name="c", subcore_axis_name="s")


@functools.partial(
    pl.kernel,
    out_type=jax.ShapeDtypeStruct((NC, N, DEGW), jnp.float32),
    mesh=_mesh,
    scratch_types=[
        pltpu.VMEM_SHARED((N, DEGW), jnp.float32),
        pltpu.VMEM((CH, DEGW), jnp.float32),
        pltpu.VMEM((CH,), jnp.int32),
    ],
    compiler_params=pltpu.CompilerParams(use_tc_tiling_on_sc=False),
)
def _deg_kernel(dst_hbm, zeros_hbm, ones_hbm, out_hbm, shared, ones_v, dst_v):
    cid = lax.axis_index("c")
    sid = lax.axis_index("s")
    wid = sid * NC + cid

    pltpu.sync_copy(ones_hbm, ones_v)
    row0 = _tile_row0(sid)
    pltpu.sync_copy(zeros_hbm.at[pl.ds(row0, RPT)], shared.at[pl.ds(row0, RPT)])
    plsc.subcore_barrier()

    def chunk(k, _):
        c = wid + k * NW

        @pl.when(c < NCHUNK)
        def _():
            pltpu.sync_copy(dst_hbm.at[pl.ds(c * CH, CH)], dst_v)
            pltpu.sync_copy(ones_v, shared.at[dst_v], add=True)

        return _

    lax.fori_loop(0, KMAX, chunk, None)
    plsc.subcore_barrier()
    pltpu.sync_copy(
        shared.at[pl.ds(row0, RPT)],
        out_hbm.at[cid, pl.ds(row0, RPT)],
    )


@functools.partial(
    pl.kernel,
    out_type=jax.ShapeDtypeStruct((NC, N, DIM), jnp.float32),
    mesh=_mesh,
    scratch_types=[
        pltpu.VMEM_SHARED((N, DIM), jnp.float32),
        pltpu.VMEM((CH,), jnp.int32),
        pltpu.VMEM((CH,), jnp.int32),
        pltpu.VMEM((CH, DIM), jnp.float32),
        pltpu.SemaphoreType.DMA,
    ],
    compiler_params=pltpu.CompilerParams(use_tc_tiling_on_sc=False),
)
def _agg_kernel(zs_hbm, src_hbm, dst_hbm, zeros_hbm, out_hbm,
                shared, src_v, dst_v, rows_v, sem):
    cid = lax.axis_index("c")
    sid = lax.axis_index("s")
    wid = sid * NC + cid

    row0 = _tile_row0(sid)
    pltpu.sync_copy(zeros_hbm.at[pl.ds(row0, RPT)],
                    shared.at[pl.ds(row0, RPT)])
    plsc.subcore_barrier()

    def chunk(k, _):
        c = wid + k * NW

        @pl.when(c < NCHUNK)
        def _():
            pltpu.sync_copy(src_hbm.at[pl.ds(c * CH, CH)], src_v)
            pltpu.sync_copy(dst_hbm.at[pl.ds(c * CH, CH)], dst_v)
            pltpu.async_copy(zs_hbm.at[src_v], rows_v, sem).wait()
            pltpu.sync_copy(rows_v, shared.at[dst_v], add=True)

        return _

    lax.fori_loop(0, KMAX, chunk, None)
    plsc.subcore_barrier()
    pltpu.sync_copy(
        shared.at[pl.ds(row0, RPT)],
        out_hbm.at[cid, pl.ds(row0, RPT)],
    )


def _tc1_body(x_ref, w1_ref, degp_ref, zs1_ref, d_ref):
    deg = degp_ref[0, :, 0:1] + degp_ref[1, :, 0:1] + 2.0
    d = lax.rsqrt(deg)
    z1 = jnp.dot(x_ref[...], w1_ref[...], preferred_element_type=jnp.float32)
    zs1_ref[...] = z1 * d
    d_ref[...] = d


def _tc_mid_body(aggp_ref, zs_ref, d_ref, b_ref, w_ref, zsn_ref):
    d = d_ref[...]
    agg = aggp_ref[0] + aggp_ref[1] + 2.0 * zs_ref[...]
    h = jnp.maximum(d * agg + b_ref[...], 0.0)
    z = jnp.dot(h, w_ref[...], preferred_element_type=jnp.float32)
    zsn_ref[...] = z * d


def _tc_out_body(aggp_ref, zs_ref, d_ref, b_ref, wl_ref, bl_ref, out_ref):
    d = d_ref[...]
    agg = aggp_ref[0] + aggp_ref[1] + 2.0 * zs_ref[...]
    h = jnp.maximum(d * agg + b_ref[...], 0.0)
    lo = jnp.dot(h, wl_ref[...], preferred_element_type=jnp.float32) + bl_ref[...]
    m = jnp.max(lo, axis=1, keepdims=True)
    e = jnp.exp(lo - m)
    out_ref[...] = e / jnp.sum(e, axis=1, keepdims=True)


def kernel(x, edge_index, W1, b1, W2, b2, Wl, bl):
    src = edge_index[0].astype(jnp.int32)
    dst = edge_index[1].astype(jnp.int32)
    zeros = jnp.zeros((N, DIM), jnp.float32)
    zeros16 = jnp.zeros((N, DEGW), jnp.float32)
    ones16 = jnp.ones((CH, DEGW), jnp.float32)

    degp = _deg_kernel(dst, zeros16, ones16)

    zs1, d = pl.pallas_call(
        _tc1_body,
        out_shape=[
            jax.ShapeDtypeStruct((N, DIM), jnp.float32),
            jax.ShapeDtypeStruct((N, 1), jnp.float32),
        ],
    )(x, W1, degp)

    agg1 = _agg_kernel(zs1, src, dst, zeros)

    zs2 = pl.pallas_call(
        _tc_mid_body,
        out_shape=jax.ShapeDtypeStruct((N, DIM), jnp.float32),
    )(agg1, zs1, d, b1.reshape(1, DIM), W2)

    agg2 = _agg_kernel(zs2, src, dst, zeros)

    out = pl.pallas_call(
        _tc_out_body,
        out_shape=jax.ShapeDtypeStruct((N, C), jnp.float32),
    )(agg2, zs2, d, b2.reshape(1, DIM), Wl, bl.reshape(1, C))
    return out


# speedup vs baseline: 25.8615x; 1.2706x over previous
import functools

import jax
import jax.numpy as jnp
from jax import lax
from jax.experimental import pallas as pl
from jax.experimental.pallas import tpu as pltpu
from jax.experimental.pallas import tpu_sc as plsc

N = 10000
E = 320000
F_IN = 128
DIM = 32
C = 10

NC = 2
NS = 16
NW = NC * NS
CH = 128
KMAX = 80
E_PAD = NW * KMAX * CH
WE = KMAX * CH
N_ACC = 10240
RPT = N_ACC // NS
DEGW = 16
DGRP = 8

_mesh = plsc.VectorSubcoreMesh(core_axis_name="c", subcore_axis_name="s")


@functools.partial(
    pl.kernel,
    out_type=jax.ShapeDtypeStruct((NC, N_ACC, DEGW), jnp.float32),
    mesh=_mesh,
    scratch_types=[
        pltpu.VMEM_SHARED((N_ACC, DEGW), jnp.float32),
        pltpu.VMEM((CH, DEGW), jnp.float32),
        pltpu.VMEM((KMAX, CH), jnp.int32),
        pltpu.SemaphoreType.DMA,
    ],
    compiler_params=pltpu.CompilerParams(use_tc_tiling_on_sc=False),
)
def _deg_kernel(dst_hbm, zeros_hbm, ones_hbm, out_hbm, shared, ones_v, dstb, sem):
    cid = lax.axis_index("c")
    sid = lax.axis_index("s")
    wid = sid * NC + cid
    row0 = sid * RPT

    pltpu.sync_copy(ones_hbm, ones_v)
    pltpu.sync_copy(dst_hbm.at[pl.ds(wid * KMAX, KMAX)], dstb)
    pltpu.sync_copy(zeros_hbm.at[pl.ds(row0, RPT)], shared.at[pl.ds(row0, RPT)])
    plsc.subcore_barrier()

    def group(g, _):
        for b in range(DGRP):
            pltpu.async_copy(ones_v, shared.at[dstb.at[g * DGRP + b]], sem,
                             add=True)
        for b in range(DGRP):
            pltpu.make_async_copy(ones_v, shared.at[dstb.at[0]], sem).wait()
        return _

    lax.fori_loop(0, KMAX // DGRP, group, None)
    plsc.subcore_barrier()
    pltpu.sync_copy(
        shared.at[pl.ds(row0, RPT)],
        out_hbm.at[cid, pl.ds(row0, RPT)],
    )


@functools.partial(
    pl.kernel,
    out_type=jax.ShapeDtypeStruct((NC, N_ACC, DIM), jnp.float32),
    mesh=_mesh,
    scratch_types=[
        pltpu.VMEM_SHARED((N_ACC, DIM), jnp.float32),
        pltpu.VMEM((KMAX, CH), jnp.int32),
        pltpu.VMEM((KMAX, CH), jnp.int32),
        pltpu.VMEM((CH, DIM), jnp.float32),
        pltpu.VMEM((CH, DIM), jnp.float32),
        pltpu.SemaphoreType.DMA,
        pltpu.SemaphoreType.DMA,
    ],
    compiler_params=pltpu.CompilerParams(use_tc_tiling_on_sc=False),
)
def _agg_kernel(zs_hbm, src_hbm, dst_hbm, zeros_hbm, out_hbm,
                shared, srcb, dstb, rows_a, rows_b, gsem_a, gsem_b):
    cid = lax.axis_index("c")
    sid = lax.axis_index("s")
    wid = sid * NC + cid
    row0 = sid * RPT

    pltpu.sync_copy(src_hbm.at[pl.ds(wid * KMAX, KMAX)], srcb)
    pltpu.sync_copy(dst_hbm.at[pl.ds(wid * KMAX, KMAX)], dstb)
    pltpu.sync_copy(zeros_hbm.at[pl.ds(row0, RPT)], shared.at[pl.ds(row0, RPT)])
    plsc.subcore_barrier()

    pltpu.async_copy(zs_hbm.at[srcb.at[0]], rows_a, gsem_a)

    def body(j, _):
        k0 = 2 * j

        pltpu.make_async_copy(zs_hbm.at[srcb.at[0]], rows_a, gsem_a).wait()
        pltpu.async_copy(zs_hbm.at[srcb.at[k0 + 1]], rows_b, gsem_b)
        pltpu.sync_copy(rows_a, shared.at[dstb.at[k0]], add=True)

        pltpu.make_async_copy(zs_hbm.at[srcb.at[0]], rows_b, gsem_b).wait()

        @pl.when(j < KMAX // 2 - 1)
        def _():
            pltpu.async_copy(zs_hbm.at[srcb.at[k0 + 2]], rows_a, gsem_a)

        pltpu.sync_copy(rows_b, shared.at[dstb.at[k0 + 1]], add=True)
        return _

    lax.fori_loop(0, KMAX // 2, body, None)
    plsc.subcore_barrier()
    pltpu.sync_copy(
        shared.at[pl.ds(row0, RPT)],
        out_hbm.at[cid, pl.ds(row0, RPT)],
    )


def _tc1_body(x_ref, w1_ref, degp_ref, zs1_ref, d_ref):
    deg = degp_ref[0, 0:N, 0:1] + degp_ref[1, 0:N, 0:1] + 2.0
    d = lax.rsqrt(deg)
    z1 = jnp.dot(x_ref[...], w1_ref[...], preferred_element_type=jnp.float32)
    zs1_ref[...] = z1 * d
    d_ref[...] = d


def _tc_mid_body(aggp_ref, zs_ref, d_ref, b_ref, w_ref, zsn_ref):
    d = d_ref[...]
    agg = aggp_ref[0, 0:N] + aggp_ref[1, 0:N] + 2.0 * zs_ref[...]
    h = jnp.maximum(d * agg + b_ref[...], 0.0)
    z = jnp.dot(h, w_ref[...], preferred_element_type=jnp.float32)
    zsn_ref[...] = z * d


def _tc_out_body(aggp_ref, zs_ref, d_ref, b_ref, wl_ref, bl_ref, out_ref):
    d = d_ref[...]
    agg = aggp_ref[0, 0:N] + aggp_ref[1, 0:N] + 2.0 * zs_ref[...]
    h = jnp.maximum(d * agg + b_ref[...], 0.0)
    lo = jnp.dot(h, wl_ref[...], preferred_element_type=jnp.float32) + bl_ref[...]
    m = jnp.max(lo, axis=1, keepdims=True)
    e = jnp.exp(lo - m)
    out_ref[...] = e / jnp.sum(e, axis=1, keepdims=True)


def kernel(x, edge_index, W1, b1, W2, b2, Wl, bl):
    src = edge_index[0].astype(jnp.int32)
    dst = edge_index[1].astype(jnp.int32)
    npad = E_PAD - E
    src_p = jnp.concatenate([src, jnp.zeros((npad,), jnp.int32)])
    dst_p = jnp.concatenate(
        [dst, N + (jnp.arange(npad, dtype=jnp.int32) % (N_ACC - N))])
    src_p = src_p.reshape(NW * KMAX, CH)
    dst_p = dst_p.reshape(NW * KMAX, CH)

    zeros32 = jnp.zeros((N_ACC, DIM), jnp.float32)
    zeros16 = jnp.zeros((N_ACC, DEGW), jnp.float32)
    ones16 = jnp.ones((CH, DEGW), jnp.float32)

    degp = _deg_kernel(dst_p, zeros16, ones16)

    zs1, d = pl.pallas_call(
        _tc1_body,
        out_shape=[
            jax.ShapeDtypeStruct((N, DIM), jnp.float32),
            jax.ShapeDtypeStruct((N, 1), jnp.float32),
        ],
    )(x, W1, degp)

    agg1 = _agg_kernel(zs1, src_p, dst_p, zeros32)

    zs2 = pl.pallas_call(
        _tc_mid_body,
        out_shape=jax.ShapeDtypeStruct((N, DIM), jnp.float32),
    )(agg1, zs1, d, b1.reshape(1, DIM), W2)

    agg2 = _agg_kernel(zs2, src_p, dst_p, zeros32)

    out = pl.pallas_call(
        _tc_out_body,
        out_shape=jax.ShapeDtypeStruct((N, C), jnp.float32),
    )(agg2, zs2, d, b2.reshape(1, DIM), Wl, bl.reshape(1, C))
    return out


# speedup vs baseline: 29.6156x; 1.1452x over previous
import functools

import jax
import jax.numpy as jnp
from jax import lax
from jax.experimental import pallas as pl
from jax.experimental.pallas import tpu as pltpu
from jax.experimental.pallas import tpu_sc as plsc

N = 10000
E = 320000
F_IN = 128
DIM = 32
C = 10

NC = 2
NS = 16
NW = NC * NS
CH = 128
KMAX = 80
E_PAD = NW * KMAX * CH
WE = KMAX * CH
N_ACC = 10240
NZ = N + 8
RPT = N_ACC // NS
DEGW = 16
DGRP = 8

_mesh = plsc.VectorSubcoreMesh(core_axis_name="c", subcore_axis_name="s")


@functools.partial(
    pl.kernel,
    out_type=jax.ShapeDtypeStruct((NC, N_ACC, DEGW), jnp.float32),
    mesh=_mesh,
    scratch_types=[
        pltpu.VMEM_SHARED((N_ACC, DEGW), jnp.float32),
        pltpu.VMEM((CH, DEGW), jnp.float32),
        pltpu.VMEM((KMAX, CH), jnp.int32),
        pltpu.SemaphoreType.DMA,
    ],
    compiler_params=pltpu.CompilerParams(use_tc_tiling_on_sc=False),
)
def _deg_kernel(dst_hbm, zeros_hbm, ones_hbm, out_hbm, shared, ones_v, dstb, sem):
    cid = lax.axis_index("c")
    sid = lax.axis_index("s")
    wid = sid * NC + cid
    row0 = sid * RPT

    pltpu.sync_copy(ones_hbm, ones_v)
    pltpu.sync_copy(dst_hbm.at[pl.ds(wid * KMAX, KMAX)], dstb)
    pltpu.sync_copy(zeros_hbm.at[pl.ds(row0, RPT)], shared.at[pl.ds(row0, RPT)])
    plsc.subcore_barrier()

    def group(g, _):
        for b in range(DGRP):
            pltpu.async_copy(ones_v, shared.at[dstb.at[g * DGRP + b]], sem,
                             add=True)
        for b in range(DGRP):
            pltpu.make_async_copy(ones_v, shared.at[dstb.at[0]], sem).wait()
        return _

    lax.fori_loop(0, KMAX // DGRP, group, None)
    plsc.subcore_barrier()
    pltpu.sync_copy(
        shared.at[pl.ds(row0, RPT)],
        out_hbm.at[cid, pl.ds(row0, RPT)],
    )


@functools.partial(
    pl.kernel,
    out_type=jax.ShapeDtypeStruct((NC, N_ACC, DIM), jnp.float32),
    mesh=_mesh,
    scratch_types=[
        pltpu.VMEM_SHARED((N_ACC, DIM), jnp.float32),
        pltpu.VMEM((KMAX, CH), jnp.int32),
        pltpu.VMEM((KMAX, CH), jnp.int32),
        [pltpu.VMEM((CH, DIM), jnp.float32)] * 4,
        [pltpu.SemaphoreType.DMA] * 4,
        [pltpu.SemaphoreType.DMA] * 4,
    ],
    compiler_params=pltpu.CompilerParams(use_tc_tiling_on_sc=False),
)
def _agg_kernel(zs_hbm, src_hbm, dst_hbm, zeros_hbm, out_hbm,
                shared, srcb, dstb, rows, gsem, ssem):
    cid = lax.axis_index("c")
    sid = lax.axis_index("s")
    wid = sid * NC + cid
    row0 = sid * RPT

    pltpu.sync_copy(src_hbm.at[pl.ds(wid * KMAX, KMAX)], srcb)
    pltpu.sync_copy(dst_hbm.at[pl.ds(wid * KMAX, KMAX)], dstb)
    pltpu.sync_copy(zeros_hbm.at[pl.ds(row0, RPT)], shared.at[pl.ds(row0, RPT)])
    plsc.subcore_barrier()

    def gwait(i):
        pltpu.make_async_copy(zs_hbm.at[srcb.at[0]], rows[i], gsem[i]).wait()

    def swait(i):
        pltpu.make_async_copy(rows[i], shared.at[dstb.at[0]], ssem[i]).wait()

    pltpu.async_copy(zs_hbm.at[srcb.at[0]], rows[0], gsem[0])
    pltpu.async_copy(zs_hbm.at[srcb.at[1]], rows[1], gsem[1])

    def body(j, _):
        k0 = 4 * j
        for i in range(4):
            k = k0 + i
            gwait(i)
            pltpu.async_copy(rows[i], shared.at[dstb.at[k]], ssem[i], add=True)
            i2 = (i + 2) % 4

            @pl.when(k + 2 < KMAX)
            def _():
                @pl.when(k >= 2)
                def _():
                    swait(i2)
                pltpu.async_copy(zs_hbm.at[srcb.at[k + 2]], rows[i2], gsem[i2])

        return _

    lax.fori_loop(0, KMAX // 4, body, None)
    for i in range(4):
        swait((KMAX - 4 + i) % 4)
    plsc.subcore_barrier()
    pltpu.sync_copy(
        shared.at[pl.ds(row0, RPT)],
        out_hbm.at[cid, pl.ds(row0, RPT)],
    )


def _tc1_body(x_ref, w1_ref, degp_ref, zs1_ref, d_ref):
    deg = degp_ref[0, 0:N, 0:1] + degp_ref[1, 0:N, 0:1] + 2.0
    d = lax.rsqrt(deg)
    z1 = jnp.dot(x_ref[...], w1_ref[...], preferred_element_type=jnp.float32)
    zs1_ref[0:N] = z1 * d
    zs1_ref[N:NZ] = jnp.zeros((NZ - N, DIM), jnp.float32)
    d_ref[...] = d


def _tc_mid_body(aggp_ref, zs_ref, d_ref, b_ref, w_ref, zsn_ref):
    d = d_ref[...]
    agg = aggp_ref[0, 0:N] + aggp_ref[1, 0:N] + 2.0 * zs_ref[0:N]
    h = jnp.maximum(d * agg + b_ref[...], 0.0)
    z = jnp.dot(h, w_ref[...], preferred_element_type=jnp.float32)
    zsn_ref[0:N] = z * d
    zsn_ref[N:NZ] = jnp.zeros((NZ - N, DIM), jnp.float32)


def _tc_out_body(aggp_ref, zs_ref, d_ref, b_ref, wl_ref, bl_ref, out_ref):
    d = d_ref[...]
    agg = aggp_ref[0, 0:N] + aggp_ref[1, 0:N] + 2.0 * zs_ref[0:N]
    h = jnp.maximum(d * agg + b_ref[...], 0.0)
    lo = jnp.dot(h, wl_ref[...], preferred_element_type=jnp.float32) + bl_ref[...]
    m = jnp.max(lo, axis=1, keepdims=True)
    e = jnp.exp(lo - m)
    out_ref[...] = e / jnp.sum(e, axis=1, keepdims=True)


def kernel(x, edge_index, W1, b1, W2, b2, Wl, bl):
    src = edge_index[0].astype(jnp.int32)
    dst = edge_index[1].astype(jnp.int32)
    npad = E_PAD - E
    src_p = jnp.concatenate([src, jnp.full((npad,), N, jnp.int32)])
    dst_p = jnp.concatenate(
        [dst, jnp.arange(npad, dtype=jnp.int32) % N_ACC])
    dst_deg = jnp.concatenate(
        [dst, N + (jnp.arange(npad, dtype=jnp.int32) % (N_ACC - N))])
    src_p = src_p.reshape(NW * KMAX, CH)
    dst_p = dst_p.reshape(NW * KMAX, CH)
    dst_deg = dst_deg.reshape(NW * KMAX, CH)

    zeros32 = jnp.zeros((N_ACC, DIM), jnp.float32)
    zeros16 = jnp.zeros((N_ACC, DEGW), jnp.float32)
    ones16 = jnp.ones((CH, DEGW), jnp.float32)

    degp = _deg_kernel(dst_deg, zeros16, ones16)

    zs1, d = pl.pallas_call(
        _tc1_body,
        out_shape=[
            jax.ShapeDtypeStruct((NZ, DIM), jnp.float32),
            jax.ShapeDtypeStruct((N, 1), jnp.float32),
        ],
    )(x, W1, degp)

    agg1 = _agg_kernel(zs1, src_p, dst_p, zeros32)

    zs2 = pl.pallas_call(
        _tc_mid_body,
        out_shape=jax.ShapeDtypeStruct((NZ, DIM), jnp.float32),
    )(agg1, zs1, d, b1.reshape(1, DIM), W2)

    agg2 = _agg_kernel(zs2, src_p, dst_p, zeros32)

    out = pl.pallas_call(
        _tc_out_body,
        out_shape=jax.ShapeDtypeStruct((N, C), jnp.float32),
    )(agg2, zs2, d, b2.reshape(1, DIM), Wl, bl.reshape(1, C))
    return out


# speedup vs baseline: 48.3437x; 1.6324x over previous
import functools

import jax
import jax.numpy as jnp
from jax import lax
from jax.experimental import pallas as pl
from jax.experimental.pallas import tpu as pltpu
from jax.experimental.pallas import tpu_sc as plsc

N = 10000
E = 320000
F_IN = 128
DIM = 32
C = 10

NC = 2
NS = 16
NW = NC * NS
CH = 128
KMAX = 80
E_PAD = NW * KMAX * CH
WE = KMAX * CH
N_ACC = 10240
NZ = N_ACC
RPT = N_ACC // NS
DEGW = 8
DGRP = 8

_mesh = plsc.VectorSubcoreMesh(core_axis_name="c", subcore_axis_name="s")


@functools.partial(
    pl.kernel,
    out_type=jax.ShapeDtypeStruct((NC, N_ACC, DEGW), jnp.float32),
    mesh=_mesh,
    scratch_types=[
        pltpu.VMEM_SHARED((N_ACC, DEGW), jnp.float32),
        pltpu.VMEM((CH, DEGW), jnp.float32),
        pltpu.VMEM((KMAX, CH), jnp.int32),
        pltpu.SemaphoreType.DMA,
    ],
    compiler_params=pltpu.CompilerParams(use_tc_tiling_on_sc=False),
)
def _deg_kernel(dst_hbm, zeros_hbm, ones_hbm, out_hbm, shared, ones_v, dstb, sem):
    cid = lax.axis_index("c")
    sid = lax.axis_index("s")
    wid = sid * NC + cid
    row0 = sid * RPT

    pltpu.sync_copy(ones_hbm, ones_v)
    pltpu.sync_copy(dst_hbm.at[pl.ds(wid * KMAX, KMAX)], dstb)
    pltpu.sync_copy(zeros_hbm.at[pl.ds(row0, RPT)], shared.at[pl.ds(row0, RPT)])
    plsc.subcore_barrier()

    def group(g, _):
        for b in range(DGRP):
            pltpu.async_copy(ones_v, shared.at[dstb.at[g * DGRP + b]], sem,
                             add=True)
        for b in range(DGRP):
            pltpu.make_async_copy(ones_v, shared.at[dstb.at[0]], sem).wait()
        return _

    lax.fori_loop(0, KMAX // DGRP, group, None)
    plsc.subcore_barrier()
    pltpu.sync_copy(
        shared.at[pl.ds(row0, RPT)],
        out_hbm.at[cid, pl.ds(row0, RPT)],
    )


@functools.partial(
    pl.kernel,
    out_type=jax.ShapeDtypeStruct((NC, N_ACC, DIM), jnp.float32),
    mesh=_mesh,
    scratch_types=[
        pltpu.VMEM_SHARED((N_ACC, DIM), jnp.float32),
        pltpu.VMEM_SHARED((NZ, DIM), jnp.float32),
        pltpu.VMEM((KMAX, CH), jnp.int32),
        pltpu.VMEM((KMAX, CH), jnp.int32),
        [pltpu.VMEM((CH, DIM), jnp.float32)] * 4,
        [pltpu.SemaphoreType.DMA] * 4,
        [pltpu.SemaphoreType.DMA] * 4,
    ],
    compiler_params=pltpu.CompilerParams(use_tc_tiling_on_sc=False),
)
def _agg_kernel(zs_hbm, src_hbm, dst_hbm, zeros_hbm, out_hbm,
                shared, zs_sp, srcb, dstb, rows, gsem, ssem):
    cid = lax.axis_index("c")
    sid = lax.axis_index("s")
    wid = sid * NC + cid
    row0 = sid * RPT

    pltpu.sync_copy(src_hbm.at[pl.ds(wid * KMAX, KMAX)], srcb)
    pltpu.sync_copy(dst_hbm.at[pl.ds(wid * KMAX, KMAX)], dstb)
    pltpu.sync_copy(zs_hbm.at[pl.ds(row0, RPT)], zs_sp.at[pl.ds(row0, RPT)])
    pltpu.sync_copy(zeros_hbm.at[pl.ds(row0, RPT)], shared.at[pl.ds(row0, RPT)])
    plsc.subcore_barrier()

    def gwait(i):
        pltpu.make_async_copy(zs_sp.at[srcb.at[0]], rows[i], gsem[i]).wait()

    def swait(i):
        pltpu.make_async_copy(rows[i], shared.at[dstb.at[0]], ssem[i]).wait()

    pltpu.async_copy(zs_sp.at[srcb.at[0]], rows[0], gsem[0])
    pltpu.async_copy(zs_sp.at[srcb.at[1]], rows[1], gsem[1])

    def body(j, _):
        k0 = 4 * j
        for i in range(4):
            k = k0 + i
            gwait(i)
            pltpu.async_copy(rows[i], shared.at[dstb.at[k]], ssem[i], add=True)
            i2 = (i + 2) % 4

            @pl.when(k + 2 < KMAX)
            def _():
                @pl.when(k >= 2)
                def _():
                    swait(i2)
                pltpu.async_copy(zs_sp.at[srcb.at[k + 2]], rows[i2], gsem[i2])

        return _

    lax.fori_loop(0, KMAX // 4, body, None)
    for i in range(4):
        swait((KMAX - 4 + i) % 4)
    plsc.subcore_barrier()
    pltpu.sync_copy(
        shared.at[pl.ds(row0, RPT)],
        out_hbm.at[cid, pl.ds(row0, RPT)],
    )


def _tc1_body(x_ref, w1_ref, degp_ref, zs1_ref, d_ref):
    deg = degp_ref[0, 0:N, 0:1] + degp_ref[1, 0:N, 0:1] + 2.0
    d = lax.rsqrt(deg)
    z1 = jnp.dot(x_ref[...], w1_ref[...], preferred_element_type=jnp.float32)
    zs1_ref[0:N] = z1 * d
    zs1_ref[N:NZ] = jnp.zeros((NZ - N, DIM), jnp.float32)
    d_ref[...] = d


def _tc_mid_body(aggp_ref, zs_ref, d_ref, b_ref, w_ref, zsn_ref):
    d = d_ref[...]
    agg = aggp_ref[0, 0:N] + aggp_ref[1, 0:N] + 2.0 * zs_ref[0:N]
    h = jnp.maximum(d * agg + b_ref[...], 0.0)
    z = jnp.dot(h, w_ref[...], preferred_element_type=jnp.float32)
    zsn_ref[0:N] = z * d
    zsn_ref[N:NZ] = jnp.zeros((NZ - N, DIM), jnp.float32)


def _tc_out_body(aggp_ref, zs_ref, d_ref, b_ref, wl_ref, bl_ref, out_ref):
    d = d_ref[...]
    agg = aggp_ref[0, 0:N] + aggp_ref[1, 0:N] + 2.0 * zs_ref[0:N]
    h = jnp.maximum(d * agg + b_ref[...], 0.0)
    lo = jnp.dot(h, wl_ref[...], preferred_element_type=jnp.float32) + bl_ref[...]
    m = jnp.max(lo, axis=1, keepdims=True)
    e = jnp.exp(lo - m)
    out_ref[...] = e / jnp.sum(e, axis=1, keepdims=True)


def kernel(x, edge_index, W1, b1, W2, b2, Wl, bl):
    src = edge_index[0].astype(jnp.int32)
    dst = edge_index[1].astype(jnp.int32)
    npad = E_PAD - E
    src_p = jnp.concatenate([src, jnp.full((npad,), N, jnp.int32)])
    dst_p = jnp.concatenate(
        [dst, jnp.arange(npad, dtype=jnp.int32) % N_ACC])
    dst_deg = jnp.concatenate(
        [dst, N + (jnp.arange(npad, dtype=jnp.int32) % (N_ACC - N))])
    src_p = src_p.reshape(NW * KMAX, CH)
    dst_p = dst_p.reshape(NW * KMAX, CH)
    dst_deg = dst_deg.reshape(NW * KMAX, CH)

    zeros32 = jnp.zeros((N_ACC, DIM), jnp.float32)
    zeros16 = jnp.zeros((N_ACC, DEGW), jnp.float32)
    ones16 = jnp.ones((CH, DEGW), jnp.float32)

    degp = _deg_kernel(dst_deg, zeros16, ones16)

    zs1, d = pl.pallas_call(
        _tc1_body,
        out_shape=[
            jax.ShapeDtypeStruct((NZ, DIM), jnp.float32),
            jax.ShapeDtypeStruct((N, 1), jnp.float32),
        ],
    )(x, W1, degp)

    agg1 = _agg_kernel(zs1, src_p, dst_p, zeros32)

    zs2 = pl.pallas_call(
        _tc_mid_body,
        out_shape=jax.ShapeDtypeStruct((NZ, DIM), jnp.float32),
    )(agg1, zs1, d, b1.reshape(1, DIM), W2)

    agg2 = _agg_kernel(zs2, src_p, dst_p, zeros32)

    out = pl.pallas_call(
        _tc_out_body,
        out_shape=jax.ShapeDtypeStruct((N, C), jnp.float32),
    )(agg2, zs2, d, b2.reshape(1, DIM), Wl, bl.reshape(1, C))
    return out
